# Initial kernel scaffold; baseline (speedup 1.0000x reference)
import jax
import jax.numpy as jnp
from jax.experimental import pallas as pl


def kernel(instance_tokens, image_tokens, img_idxs, layout_masks, alpha, W_lh, b_lh, W_q, W_out):
    raise NotImplementedError("write your pallas kernel here")


# speedup vs baseline: 25.7460x; 25.7460x over previous
import functools

import jax
import jax.numpy as jnp
from jax import lax
from jax.experimental import pallas as pl
from jax.experimental.pallas import tpu as pltpu

B, R, N, L, D, H, Dh = 4, 8, 2048, 256, 3072, 24, 128
BR = B * R
D2 = 2 * D

E_BLK = 512
O_BLK = 512
N_BLK = 256
NB = N // N_BLK


def _layout_kv_body(x_ref, w_ref, b_ref, alpha_ref, mask_ref,
                    lo_ref, kvs_ref):
    x = x_ref[...]
    w = w_ref[...]
    kv = lax.dot_general(x, w, (((1,), (1,)), ((), ())),
                         preferred_element_type=jnp.float32)
    kv = kv + b_ref[...]
    valid = mask_ref[...] == 1.0
    lo_ref[...] = jnp.where(valid, kv, 0.0)
    kvs_ref[...] = kv * (1.0 - alpha_ref[...])


def _out_vec_body(v_ref, w_ref, o_ref):
    o_ref[...] = lax.dot_general(v_ref[...], w_ref[...],
                                 (((1,), (1,)), ((), ())),
                                 preferred_element_type=jnp.float32)


def _assemble_body(img_ref, idx_ref, mask_ref, ov_ref, out_ref):
    nb = pl.program_id(0) % NB
    n0 = nb * N_BLK
    ids = n0 + lax.broadcasted_iota(jnp.int32, (N_BLK, L), 0)
    winner = jnp.full((N_BLK, 1), -1, dtype=jnp.int32)
    for j in range(R):
        idx_j = idx_ref[0, j, :][None, :]
        hit = jnp.any(idx_j == ids, axis=1, keepdims=True)
        valid = mask_ref[0, j, 0] == 1.0
        winner = jnp.where(hit & valid, j, winner)
    onehot = (winner == lax.broadcasted_iota(jnp.int32, (N_BLK, R), 1))
    rows = lax.dot_general(onehot.astype(jnp.float32), ov_ref[0],
                           (((1,), (0,)), ((), ())),
                           preferred_element_type=jnp.float32)
    out_ref[0] = jnp.where(winner >= 0, rows, img_ref[0])


def kernel(instance_tokens, image_tokens, img_idxs, layout_masks, alpha,
           W_lh, b_lh, W_q, W_out):
    x = instance_tokens.reshape(BR, D)
    alpha2 = alpha.reshape(BR, 1)
    mask2 = layout_masks.reshape(BR, 1)
    b_lh2 = b_lh.reshape(1, D2)

    lo, kvs = pl.pallas_call(
        _layout_kv_body,
        grid=(D2 // E_BLK,),
        in_specs=[
            pl.BlockSpec((BR, D), lambda e: (0, 0)),
            pl.BlockSpec((E_BLK, D), lambda e: (e, 0)),
            pl.BlockSpec((1, E_BLK), lambda e: (0, e)),
            pl.BlockSpec((BR, 1), lambda e: (0, 0)),
            pl.BlockSpec((BR, 1), lambda e: (0, 0)),
        ],
        out_specs=[
            pl.BlockSpec((BR, E_BLK), lambda e: (0, e)),
            pl.BlockSpec((BR, E_BLK), lambda e: (0, e)),
        ],
        out_shape=[
            jax.ShapeDtypeStruct((BR, D2), jnp.float32),
            jax.ShapeDtypeStruct((BR, D2), jnp.float32),
        ],
    )(x, W_lh, b_lh2, alpha2, mask2)

    v_scaled = kvs[:, D:]

    out_vec = pl.pallas_call(
        _out_vec_body,
        grid=(D // O_BLK,),
        in_specs=[
            pl.BlockSpec((BR, D), lambda o: (0, 0)),
            pl.BlockSpec((O_BLK, D), lambda o: (o, 0)),
        ],
        out_specs=pl.BlockSpec((BR, O_BLK), lambda o: (0, o)),
        out_shape=jax.ShapeDtypeStruct((BR, D), jnp.float32),
    )(v_scaled, W_out)

    ov3 = out_vec.reshape(B, R, D)
    mask3 = layout_masks.reshape(B, R, 1)

    updated = pl.pallas_call(
        _assemble_body,
        grid=(B * NB,),
        in_specs=[
            pl.BlockSpec((1, N_BLK, D), lambda i: (i // NB, i % NB, 0)),
            pl.BlockSpec((1, R, L), lambda i: (i // NB, 0, 0)),
            pl.BlockSpec((1, R, 1), lambda i: (i // NB, 0, 0)),
            pl.BlockSpec((1, R, D), lambda i: (i // NB, 0, 0)),
        ],
        out_specs=pl.BlockSpec((1, N_BLK, D), lambda i: (i // NB, i % NB, 0)),
        out_shape=jax.ShapeDtypeStruct((B, N, D), jnp.float32),
    )(image_tokens, img_idxs, mask3, ov3)

    layout_outputs = lo.reshape(B, R, D2)
    return updated, layout_outputs
